# baseline (speedup 1.0000x reference)
import jax
from jax import lax
import jax.numpy as jnp
from jax.experimental import pallas as pl
from jax.experimental.pallas import tpu as pltpu
from jax.experimental.pallas import tpu_sc as plsc

D_MODEL = 64
SCALE = 8.0
GW = 128
NBUF = 5
HALF = 2
NW = 32
VC = 4096
BB = 512


def _stage_body(t_ref, o_ref):
    o_ref[...] = (t_ref[...] * SCALE).T


def _gather_body(t2_hbm, idx_hbm, g_hbm, idx_v, rows, gsem, wsem):
    steps = idx_hbm.shape[1]
    wid = lax.axis_index("c") * 16 + lax.axis_index("s")
    base = wid * (steps * GW)

    pltpu.sync_copy(idx_hbm.at[wid], idx_v)

    def g(j, b):
        return pltpu.make_async_copy(
            t2_hbm.at[idx_v.at[j]], rows.at[b], gsem.at[b])

    def wb(j, b):
        return pltpu.make_async_copy(
            rows.at[b], g_hbm.at[pl.ds(base + j * GW, GW)], wsem.at[b])

    for j in range(HALF):
        g(j, j % NBUF).start()
    for j in range(HALF, NBUF):
        g(j, j % NBUF).start()
        g(j - HALF, (j - HALF) % NBUF).wait()
        wb(j - HALF, (j - HALF) % NBUF).start()

    @pl.loop(NBUF, steps, step=NBUF)
    def _(v):
        for dj in range(NBUF):
            j = v + dj
            b = dj
            bh = (dj + NBUF - HALF) % NBUF
            wb(j - NBUF, b).wait()
            g(j, b).start()
            g(j - HALF, bh).wait()
            wb(j - HALF, bh).start()

    for j in range(steps, steps + HALF):
        bh = (j - HALF) % NBUF
        g(j - HALF, bh).wait()
        wb(j - HALF, bh).start()
    for j in range(steps - NBUF, steps):
        wb(j, j % NBUF).wait()


def _out_body(g_ref, o_ref):
    o_ref[0] = g_ref[...].T


def kernel(x, table):
    b, p = x.shape
    v = table.shape[0]
    n = b * p

    t2 = pl.pallas_call(
        _stage_body,
        grid=(pl.cdiv(v, VC),),
        in_specs=[pl.BlockSpec((D_MODEL, VC), lambda k: (0, k))],
        out_specs=pl.BlockSpec((VC, D_MODEL), lambda k: (k, 0)),
        out_shape=jax.ShapeDtypeStruct((v, D_MODEL), jnp.float32),
    )(table.T)

    steps = n // (NW * GW)
    idx3 = x.T.reshape(-1).astype(jnp.int32).reshape(NW, steps, GW)
    mesh = plsc.VectorSubcoreMesh(core_axis_name="c", subcore_axis_name="s")
    g = pl.kernel(
        _gather_body,
        out_type=jax.ShapeDtypeStruct((n, D_MODEL), jnp.float32),
        mesh=mesh,
        scratch_types=[
            pltpu.VMEM((steps, GW), jnp.int32),
            pltpu.VMEM((NBUF, GW, D_MODEL), jnp.float32),
            pltpu.SemaphoreType.DMA((NBUF,)),
            pltpu.SemaphoreType.DMA((NBUF,)),
        ],
        compiler_params=pltpu.CompilerParams(use_tc_tiling_on_sc=False),
    )(t2, idx3)

    o3 = pl.pallas_call(
        _out_body,
        grid=(p, b // BB),
        in_specs=[pl.BlockSpec((BB, D_MODEL),
                               lambda i, c: (i * (16384 // BB) + c, 0))],
        out_specs=pl.BlockSpec((1, D_MODEL, BB), lambda i, c: (i, 0, c)),
        out_shape=jax.ShapeDtypeStruct((p, D_MODEL, b), jnp.float32),
    )(g)

    return o3.transpose(2, 0, 1)


# speedup vs baseline: 1.4110x; 1.4110x over previous
import jax
from jax import lax
import jax.numpy as jnp
from jax.experimental import pallas as pl
from jax.experimental.pallas import tpu as pltpu
from jax.experimental.pallas import tpu_sc as plsc

D_MODEL = 64
SCALE = 8.0
GW = 128
NBUF = 5
HALF = 2
NW = 32
VB = 4096
BB = 512


def _stage_body(a_ref, b_ref, o_ref):
    o_ref[:, :D_MODEL] = (a_ref[...] * SCALE).T
    o_ref[:, D_MODEL:] = (b_ref[...] * SCALE).T


def _gather_body(t2_hbm, idx_hbm, g_hbm, idx_v, rows, gsem, wsem):
    steps = idx_hbm.shape[1]
    wid = lax.axis_index("c") * 16 + lax.axis_index("s")
    base = wid * (steps * GW)

    pltpu.sync_copy(idx_hbm.at[wid], idx_v)

    def g(j, b):
        return pltpu.make_async_copy(
            t2_hbm.at[idx_v.at[j]], rows.at[b], gsem.at[b])

    def wb(j, b):
        return pltpu.make_async_copy(
            rows.at[b], g_hbm.at[pl.ds(base + j * GW, GW)], wsem.at[b])

    for j in range(HALF):
        g(j, j % NBUF).start()
    for j in range(HALF, NBUF):
        g(j, j % NBUF).start()
        g(j - HALF, (j - HALF) % NBUF).wait()
        wb(j - HALF, (j - HALF) % NBUF).start()

    @pl.loop(NBUF, steps, step=NBUF)
    def _(v):
        for dj in range(NBUF):
            j = v + dj
            b = dj
            bh = (dj + NBUF - HALF) % NBUF
            wb(j - NBUF, b).wait()
            g(j, b).start()
            g(j - HALF, bh).wait()
            wb(j - HALF, bh).start()

    for j in range(steps, steps + HALF):
        bh = (j - HALF) % NBUF
        g(j - HALF, bh).wait()
        wb(j - HALF, bh).start()
    for j in range(steps - NBUF, steps):
        wb(j, j % NBUF).wait()


def _out_body(g_ref, o_ref):
    blk = g_ref[...]
    o_ref[0, :, :BB // 2] = blk[:, :D_MODEL].T
    o_ref[0, :, BB // 2:] = blk[:, D_MODEL:].T


def kernel(x, table):
    b, p = x.shape
    v = table.shape[0]
    n = b * p

    nk = pl.cdiv(v, 2 * VB)
    last = pl.cdiv(v, VB) - 1
    t2p = pl.pallas_call(
        _stage_body,
        grid=(nk,),
        in_specs=[
            pl.BlockSpec((D_MODEL, VB), lambda k: (0, k)),
            pl.BlockSpec((D_MODEL, VB),
                         lambda k: (0, jnp.minimum(k + nk, last))),
        ],
        out_specs=pl.BlockSpec((VB, 2 * D_MODEL), lambda k: (k, 0)),
        out_shape=jax.ShapeDtypeStruct((nk * VB, 2 * D_MODEL), jnp.float32),
    )(table.T, table.T)
    t2 = t2p.reshape(2 * nk * VB, D_MODEL)

    steps = n // (NW * GW)
    xt = x.T.astype(jnp.int32)
    xt = xt.reshape(p, b // BB, 2, BB // 2).transpose(0, 1, 3, 2)
    nk = pl.cdiv(table.shape[0], 2 * VB)
    q = xt >> 12
    j = xt & 4095
    h = (q >= nk).astype(jnp.int32)
    vprime = ((q - nk * h) << 13) + (j << 1) + h
    idx3 = vprime.reshape(NW, steps, GW)
    mesh = plsc.VectorSubcoreMesh(core_axis_name="c", subcore_axis_name="s")
    g = pl.kernel(
        _gather_body,
        out_type=jax.ShapeDtypeStruct((n, D_MODEL), jnp.float32),
        mesh=mesh,
        scratch_types=[
            pltpu.VMEM((steps, GW), jnp.int32),
            pltpu.VMEM((NBUF, GW, D_MODEL), jnp.float32),
            pltpu.SemaphoreType.DMA((NBUF,)),
            pltpu.SemaphoreType.DMA((NBUF,)),
        ],
        compiler_params=pltpu.CompilerParams(use_tc_tiling_on_sc=False),
    )(t2, idx3)
    gp = g.reshape(n // 2, 2 * D_MODEL)

    o3 = pl.pallas_call(
        _out_body,
        grid=(p, b // BB),
        in_specs=[pl.BlockSpec((BB // 2, 2 * D_MODEL),
                               lambda i, c: (i * (16384 // BB) + c, 0))],
        out_specs=pl.BlockSpec((1, D_MODEL, BB), lambda i, c: (i, 0, c)),
        out_shape=jax.ShapeDtypeStruct((p, D_MODEL, b), jnp.float32),
    )(gp)

    return o3.transpose(2, 0, 1)


# speedup vs baseline: 2.0096x; 1.4243x over previous
import jax
from jax import lax
import jax.numpy as jnp
from jax.experimental import pallas as pl
from jax.experimental.pallas import tpu as pltpu
from jax.experimental.pallas import tpu_sc as plsc

D_MODEL = 64
SCALE = 8.0
GW = 128
NBUF = 5
HALF = 2
NW = 32
VB = 8192
BB = 2048


def _stage_body(a_ref, b_ref, o_ref):
    o_ref[:, :D_MODEL] = (a_ref[...] * SCALE).T
    o_ref[:, D_MODEL:] = (b_ref[...] * SCALE).T


def _gather_body(t2_hbm, idx_hbm, g_hbm, idx_v, rows, gsem, wsem):
    steps = idx_hbm.shape[1]
    wid = lax.axis_index("c") * 16 + lax.axis_index("s")
    base = wid * (steps * GW)

    pltpu.sync_copy(idx_hbm.at[wid], idx_v)

    def g(j, b):
        return pltpu.make_async_copy(
            t2_hbm.at[idx_v.at[j]], rows.at[b], gsem.at[b])

    def wb(j, b):
        return pltpu.make_async_copy(
            rows.at[b], g_hbm.at[pl.ds(base + j * GW, GW)], wsem.at[b])

    for j in range(HALF):
        g(j, j % NBUF).start()
    for j in range(HALF, NBUF):
        g(j, j % NBUF).start()
        g(j - HALF, (j - HALF) % NBUF).wait()
        wb(j - HALF, (j - HALF) % NBUF).start()

    @pl.loop(NBUF, steps, step=NBUF)
    def _(v):
        for dj in range(NBUF):
            j = v + dj
            b = dj
            bh = (dj + NBUF - HALF) % NBUF
            wb(j - NBUF, b).wait()
            g(j, b).start()
            g(j - HALF, bh).wait()
            wb(j - HALF, bh).start()

    for j in range(steps, steps + HALF):
        bh = (j - HALF) % NBUF
        g(j - HALF, bh).wait()
        wb(j - HALF, bh).start()
    for j in range(steps - NBUF, steps):
        wb(j, j % NBUF).wait()


def _out_body(g_ref, o_ref):
    blk = g_ref[...]
    o_ref[0, :, :BB // 2] = blk[:, :D_MODEL].T
    o_ref[0, :, BB // 2:] = blk[:, D_MODEL:].T


def kernel(x, table):
    b, p = x.shape
    v = table.shape[0]
    n = b * p

    nk = pl.cdiv(v, 2 * VB)
    last = pl.cdiv(v, VB) - 1
    t2p = pl.pallas_call(
        _stage_body,
        grid=(nk,),
        in_specs=[
            pl.BlockSpec((D_MODEL, VB), lambda k: (0, k)),
            pl.BlockSpec((D_MODEL, VB),
                         lambda k: (0, jnp.minimum(k + nk, last))),
        ],
        out_specs=pl.BlockSpec((VB, 2 * D_MODEL), lambda k: (k, 0)),
        out_shape=jax.ShapeDtypeStruct((nk * VB, 2 * D_MODEL), jnp.float32),
    )(table.T, table.T)
    t2 = t2p.reshape(2 * nk * VB, D_MODEL)

    steps = n // (NW * GW)
    xt = x.T.astype(jnp.int32)
    xt = xt.reshape(p, b // BB, 2, BB // 2).transpose(0, 1, 3, 2)
    nk = pl.cdiv(table.shape[0], 2 * VB)
    sh = VB.bit_length() - 1
    q = xt >> sh
    j = xt & (VB - 1)
    h = (q >= nk).astype(jnp.int32)
    vprime = ((q - nk * h) << (sh + 1)) + (j << 1) + h
    idx3 = vprime.reshape(NW, steps, GW)
    mesh = plsc.VectorSubcoreMesh(core_axis_name="c", subcore_axis_name="s")
    g = pl.kernel(
        _gather_body,
        out_type=jax.ShapeDtypeStruct((n, D_MODEL), jnp.float32),
        mesh=mesh,
        scratch_types=[
            pltpu.VMEM((steps, GW), jnp.int32),
            pltpu.VMEM((NBUF, GW, D_MODEL), jnp.float32),
            pltpu.SemaphoreType.DMA((NBUF,)),
            pltpu.SemaphoreType.DMA((NBUF,)),
        ],
        compiler_params=pltpu.CompilerParams(use_tc_tiling_on_sc=False),
    )(t2, idx3)
    gp = g.reshape(n // 2, 2 * D_MODEL)

    o3 = pl.pallas_call(
        _out_body,
        grid=(p, b // BB),
        in_specs=[pl.BlockSpec((BB // 2, 2 * D_MODEL),
                               lambda i, c: (i * (16384 // BB) + c, 0))],
        out_specs=pl.BlockSpec((1, D_MODEL, BB), lambda i, c: (i, 0, c)),
        out_shape=jax.ShapeDtypeStruct((p, D_MODEL, b), jnp.float32),
    )(gp)

    return o3.transpose(2, 0, 1)


# speedup vs baseline: 2.1847x; 1.0872x over previous
import jax
from jax import lax
import jax.numpy as jnp
from jax.experimental import pallas as pl
from jax.experimental.pallas import tpu as pltpu
from jax.experimental.pallas import tpu_sc as plsc

D_MODEL = 64
SCALE = 8.0
GW = 128
NBUF = 5
HALF = 2
NW = 32
VB = 16384
BB = 4096


def _stage_body(a_ref, b_ref, o_ref):
    o_ref[:, :D_MODEL] = (a_ref[...] * SCALE).T
    o_ref[:, D_MODEL:] = (b_ref[...] * SCALE).T


def _gather_body(t2_hbm, idx_hbm, g_hbm, idx_v, rows, gsem, wsem):
    steps = idx_hbm.shape[1]
    wid = lax.axis_index("c") * 16 + lax.axis_index("s")
    base = wid * (steps * GW)

    pltpu.sync_copy(idx_hbm.at[wid], idx_v)

    def g(j, b):
        return pltpu.make_async_copy(
            t2_hbm.at[idx_v.at[j]], rows.at[b], gsem.at[b])

    def wb(j, b):
        return pltpu.make_async_copy(
            rows.at[b], g_hbm.at[pl.ds(base + j * GW, GW)], wsem.at[b])

    for j in range(HALF):
        g(j, j % NBUF).start()
    for j in range(HALF, NBUF):
        g(j, j % NBUF).start()
        g(j - HALF, (j - HALF) % NBUF).wait()
        wb(j - HALF, (j - HALF) % NBUF).start()

    @pl.loop(NBUF, steps, step=NBUF)
    def _(v):
        for dj in range(NBUF):
            j = v + dj
            b = dj
            bh = (dj + NBUF - HALF) % NBUF
            wb(j - NBUF, b).wait()
            g(j, b).start()
            g(j - HALF, bh).wait()
            wb(j - HALF, bh).start()

    for j in range(steps, steps + HALF):
        bh = (j - HALF) % NBUF
        g(j - HALF, bh).wait()
        wb(j - HALF, bh).start()
    for j in range(steps - NBUF, steps):
        wb(j, j % NBUF).wait()


def _out_body(g_ref, o_ref):
    blk = g_ref[...]
    o_ref[0, :, :BB // 2] = blk[:, :D_MODEL].T
    o_ref[0, :, BB // 2:] = blk[:, D_MODEL:].T


def kernel(x, table):
    b, p = x.shape
    v = table.shape[0]
    n = b * p

    nk = pl.cdiv(v, 2 * VB)
    last = pl.cdiv(v, VB) - 1
    t2p = pl.pallas_call(
        _stage_body,
        grid=(nk,),
        in_specs=[
            pl.BlockSpec((D_MODEL, VB), lambda k: (0, k)),
            pl.BlockSpec((D_MODEL, VB),
                         lambda k: (0, jnp.minimum(k + nk, last))),
        ],
        out_specs=pl.BlockSpec((VB, 2 * D_MODEL), lambda k: (k, 0)),
        out_shape=jax.ShapeDtypeStruct((nk * VB, 2 * D_MODEL), jnp.float32),
    )(table.T, table.T)
    t2 = t2p.reshape(2 * nk * VB, D_MODEL)

    steps = n // (NW * GW)
    xt = x.T.astype(jnp.int32)
    xt = xt.reshape(p, b // BB, 2, BB // 2).transpose(0, 1, 3, 2)
    nk = pl.cdiv(table.shape[0], 2 * VB)
    sh = VB.bit_length() - 1
    q = xt >> sh
    j = xt & (VB - 1)
    h = (q >= nk).astype(jnp.int32)
    vprime = ((q - nk * h) << (sh + 1)) + (j << 1) + h
    idx3 = vprime.reshape(NW, steps, GW)
    mesh = plsc.VectorSubcoreMesh(core_axis_name="c", subcore_axis_name="s")
    g = pl.kernel(
        _gather_body,
        out_type=jax.ShapeDtypeStruct((n, D_MODEL), jnp.float32),
        mesh=mesh,
        scratch_types=[
            pltpu.VMEM((steps, GW), jnp.int32),
            pltpu.VMEM((NBUF, GW, D_MODEL), jnp.float32),
            pltpu.SemaphoreType.DMA((NBUF,)),
            pltpu.SemaphoreType.DMA((NBUF,)),
        ],
        compiler_params=pltpu.CompilerParams(use_tc_tiling_on_sc=False),
    )(t2, idx3)
    gp = g.reshape(n // 2, 2 * D_MODEL)

    o3 = pl.pallas_call(
        _out_body,
        grid=(p, b // BB),
        in_specs=[pl.BlockSpec((BB // 2, 2 * D_MODEL),
                               lambda i, c: (i * (16384 // BB) + c, 0))],
        out_specs=pl.BlockSpec((1, D_MODEL, BB), lambda i, c: (i, 0, c)),
        out_shape=jax.ShapeDtypeStruct((p, D_MODEL, b), jnp.float32),
    )(gp)

    return o3.transpose(2, 0, 1)


# speedup vs baseline: 2.2574x; 1.0333x over previous
import jax
from jax import lax
import jax.numpy as jnp
from jax.experimental import pallas as pl
from jax.experimental.pallas import tpu as pltpu
from jax.experimental.pallas import tpu_sc as plsc

D_MODEL = 64
SCALE = 8.0
GW = 128
NBUF = 5
HALF = 2
NW = 32
VB = 16384
BB = 8192


def _stage_body(a_ref, b_ref, o_ref):
    o_ref[:, :D_MODEL] = (a_ref[...] * SCALE).T
    o_ref[:, D_MODEL:] = (b_ref[...] * SCALE).T


def _gather_body(t2_hbm, idx_hbm, g_hbm, idx_v, rows, gsem, wsem):
    steps = idx_hbm.shape[1]
    wid = lax.axis_index("c") * 16 + lax.axis_index("s")
    base = wid * (steps * GW)

    pltpu.sync_copy(idx_hbm.at[wid], idx_v)

    def g(j, b):
        return pltpu.make_async_copy(
            t2_hbm.at[idx_v.at[j]], rows.at[b], gsem.at[b])

    def wb(j, b):
        return pltpu.make_async_copy(
            rows.at[b], g_hbm.at[pl.ds(base + j * GW, GW)], wsem.at[b])

    for j in range(HALF):
        g(j, j % NBUF).start()
    for j in range(HALF, NBUF):
        g(j, j % NBUF).start()
        g(j - HALF, (j - HALF) % NBUF).wait()
        wb(j - HALF, (j - HALF) % NBUF).start()

    @pl.loop(NBUF, steps, step=NBUF)
    def _(v):
        for dj in range(NBUF):
            j = v + dj
            b = dj
            bh = (dj + NBUF - HALF) % NBUF
            wb(j - NBUF, b).wait()
            g(j, b).start()
            g(j - HALF, bh).wait()
            wb(j - HALF, bh).start()

    for j in range(steps, steps + HALF):
        bh = (j - HALF) % NBUF
        g(j - HALF, bh).wait()
        wb(j - HALF, bh).start()
    for j in range(steps - NBUF, steps):
        wb(j, j % NBUF).wait()


def _out_body(g_ref, o_ref):
    blk = g_ref[...]
    o_ref[0, :, :BB // 2] = blk[:, :D_MODEL].T
    o_ref[0, :, BB // 2:] = blk[:, D_MODEL:].T


def kernel(x, table):
    b, p = x.shape
    v = table.shape[0]
    n = b * p

    nk = pl.cdiv(v, 2 * VB)
    last = pl.cdiv(v, VB) - 1
    t2p = pl.pallas_call(
        _stage_body,
        grid=(nk,),
        in_specs=[
            pl.BlockSpec((D_MODEL, VB), lambda k: (0, k)),
            pl.BlockSpec((D_MODEL, VB),
                         lambda k: (0, jnp.minimum(k + nk, last))),
        ],
        out_specs=pl.BlockSpec((VB, 2 * D_MODEL), lambda k: (k, 0)),
        out_shape=jax.ShapeDtypeStruct((nk * VB, 2 * D_MODEL), jnp.float32),
    )(table.T, table.T)
    t2 = t2p.reshape(2 * nk * VB, D_MODEL)

    steps = n // (NW * GW)
    xt = x.T.astype(jnp.int32)
    xt = xt.reshape(p, b // BB, 2, BB // 2).transpose(0, 1, 3, 2)
    nk = pl.cdiv(table.shape[0], 2 * VB)
    sh = VB.bit_length() - 1
    q = xt >> sh
    j = xt & (VB - 1)
    h = (q >= nk).astype(jnp.int32)
    vprime = ((q - nk * h) << (sh + 1)) + (j << 1) + h
    idx3 = vprime.reshape(NW, steps, GW)
    mesh = plsc.VectorSubcoreMesh(core_axis_name="c", subcore_axis_name="s")
    g = pl.kernel(
        _gather_body,
        out_type=jax.ShapeDtypeStruct((n, D_MODEL), jnp.float32),
        mesh=mesh,
        scratch_types=[
            pltpu.VMEM((steps, GW), jnp.int32),
            pltpu.VMEM((NBUF, GW, D_MODEL), jnp.float32),
            pltpu.SemaphoreType.DMA((NBUF,)),
            pltpu.SemaphoreType.DMA((NBUF,)),
        ],
        compiler_params=pltpu.CompilerParams(use_tc_tiling_on_sc=False),
    )(t2, idx3)
    gp = g.reshape(n // 2, 2 * D_MODEL)

    o3 = pl.pallas_call(
        _out_body,
        grid=(p, b // BB),
        in_specs=[pl.BlockSpec((BB // 2, 2 * D_MODEL),
                               lambda i, c: (i * (16384 // BB) + c, 0))],
        out_specs=pl.BlockSpec((1, D_MODEL, BB), lambda i, c: (i, 0, c)),
        out_shape=jax.ShapeDtypeStruct((p, D_MODEL, b), jnp.float32),
    )(gp)

    return o3.transpose(2, 0, 1)


# speedup vs baseline: 2.3029x; 1.0202x over previous
import jax
from jax import lax
import jax.numpy as jnp
from jax.experimental import pallas as pl
from jax.experimental.pallas import tpu as pltpu
from jax.experimental.pallas import tpu_sc as plsc

D_MODEL = 64
SCALE = 8.0
GW = 128
NBUF = 5
HALF = 2
NW = 32
VB = 16384
BB = 16384


def _stage_body(a_ref, b_ref, o_ref):
    o_ref[:, :D_MODEL] = (a_ref[...] * SCALE).T
    o_ref[:, D_MODEL:] = (b_ref[...] * SCALE).T


def _gather_body(t2_hbm, idx_hbm, g_hbm, idx_v, rows, gsem, wsem):
    steps = idx_hbm.shape[1]
    wid = lax.axis_index("c") * 16 + lax.axis_index("s")
    base = wid * (steps * GW)

    pltpu.sync_copy(idx_hbm.at[wid], idx_v)

    def g(j, b):
        return pltpu.make_async_copy(
            t2_hbm.at[idx_v.at[j]], rows.at[b], gsem.at[b])

    def wb(j, b):
        return pltpu.make_async_copy(
            rows.at[b], g_hbm.at[pl.ds(base + j * GW, GW)], wsem.at[b])

    for j in range(HALF):
        g(j, j % NBUF).start()
    for j in range(HALF, NBUF):
        g(j, j % NBUF).start()
        g(j - HALF, (j - HALF) % NBUF).wait()
        wb(j - HALF, (j - HALF) % NBUF).start()

    @pl.loop(NBUF, steps, step=NBUF)
    def _(v):
        for dj in range(NBUF):
            j = v + dj
            b = dj
            bh = (dj + NBUF - HALF) % NBUF
            wb(j - NBUF, b).wait()
            g(j, b).start()
            g(j - HALF, bh).wait()
            wb(j - HALF, bh).start()

    for j in range(steps, steps + HALF):
        bh = (j - HALF) % NBUF
        g(j - HALF, bh).wait()
        wb(j - HALF, bh).start()
    for j in range(steps - NBUF, steps):
        wb(j, j % NBUF).wait()


def _out_body(g_ref, o_ref):
    blk = g_ref[...]
    o_ref[0, :, :BB // 2] = blk[:, :D_MODEL].T
    o_ref[0, :, BB // 2:] = blk[:, D_MODEL:].T


def kernel(x, table):
    b, p = x.shape
    v = table.shape[0]
    n = b * p

    nk = pl.cdiv(v, 2 * VB)
    last = pl.cdiv(v, VB) - 1
    t2p = pl.pallas_call(
        _stage_body,
        grid=(nk,),
        in_specs=[
            pl.BlockSpec((D_MODEL, VB), lambda k: (0, k)),
            pl.BlockSpec((D_MODEL, VB),
                         lambda k: (0, jnp.minimum(k + nk, last))),
        ],
        out_specs=pl.BlockSpec((VB, 2 * D_MODEL), lambda k: (k, 0)),
        out_shape=jax.ShapeDtypeStruct((nk * VB, 2 * D_MODEL), jnp.float32),
    )(table.T, table.T)
    t2 = t2p.reshape(2 * nk * VB, D_MODEL)

    steps = n // (NW * GW)
    xt = x.T.astype(jnp.int32)
    xt = xt.reshape(p, b // BB, 2, BB // 2).transpose(0, 1, 3, 2)
    nk = pl.cdiv(table.shape[0], 2 * VB)
    sh = VB.bit_length() - 1
    q = xt >> sh
    j = xt & (VB - 1)
    h = (q >= nk).astype(jnp.int32)
    vprime = ((q - nk * h) << (sh + 1)) + (j << 1) + h
    idx3 = vprime.reshape(NW, steps, GW)
    mesh = plsc.VectorSubcoreMesh(core_axis_name="c", subcore_axis_name="s")
    g = pl.kernel(
        _gather_body,
        out_type=jax.ShapeDtypeStruct((n, D_MODEL), jnp.float32),
        mesh=mesh,
        scratch_types=[
            pltpu.VMEM((steps, GW), jnp.int32),
            pltpu.VMEM((NBUF, GW, D_MODEL), jnp.float32),
            pltpu.SemaphoreType.DMA((NBUF,)),
            pltpu.SemaphoreType.DMA((NBUF,)),
        ],
        compiler_params=pltpu.CompilerParams(use_tc_tiling_on_sc=False),
    )(t2, idx3)
    gp = g.reshape(n // 2, 2 * D_MODEL)

    o3 = pl.pallas_call(
        _out_body,
        grid=(p, b // BB),
        in_specs=[pl.BlockSpec((BB // 2, 2 * D_MODEL),
                               lambda i, c: (i * (16384 // BB) + c, 0))],
        out_specs=pl.BlockSpec((1, D_MODEL, BB), lambda i, c: (i, 0, c)),
        out_shape=jax.ShapeDtypeStruct((p, D_MODEL, b), jnp.float32),
    )(gp)

    return o3.transpose(2, 0, 1)
